# baseline (speedup 1.0000x reference)
import functools

import jax
import jax.numpy as jnp
from jax import lax
from jax.experimental import pallas as pl
from jax.experimental.pallas import tpu as pltpu
from jax.experimental.pallas import tpu_sc as plsc

_K = 1024
_D = 64
_B = 2048
_COMMITMENT_COST = 1.0

_NC = 2
_NS = 16
_NW = _NC * _NS
_CHUNK = 128
_CPW = 16


def _vq_tc(x_ref, e_ref, q_ref, idx_ref, loss_ref):
    i = pl.program_id(0)
    x = x_ref[:]
    e = e_ref[:]
    xn = jnp.sum(x * x, axis=1, keepdims=True)
    en = jnp.sum(e * e, axis=1)
    prod = jax.lax.dot_general(
        x, e, (((1,), (1,)), ((), ())), preferred_element_type=jnp.float32
    )
    dist = xn + en[None, :] - 2.0 * prod
    m = jnp.min(dist, axis=1)
    iota = jax.lax.broadcasted_iota(jnp.int32, (_B, _K), 1)
    cand = jnp.where(dist == m[:, None], iota, _K)
    idx = jnp.min(cand, axis=1).astype(jnp.int32)
    idx_ref[:] = idx
    oh = (idx[:, None] == iota).astype(jnp.float32)
    q_ref[:] = jax.lax.dot_general(
        oh, e, (((1,), (0,)), ((), ())), preferred_element_type=jnp.float32
    )

    part = jnp.sum(m)

    @pl.when(i == 0)
    def _init():
        loss_ref[0, 0] = 0.0

    loss_ref[0, 0] += part

    @pl.when(i == pl.num_programs(0) - 1)
    def _fini():
        loss_ref[0, 0] = loss_ref[0, 0] / (pl.num_programs(0) * _B * _D)


def _tc_part(x, embedding):
    n = x.shape[0]
    q, idx, loss = pl.pallas_call(
        _vq_tc,
        grid=(n // _B,),
        in_specs=[
            pl.BlockSpec((_B, _D), lambda i: (i, 0)),
            pl.BlockSpec((_K, _D), lambda i: (0, 0)),
        ],
        out_specs=[
            pl.BlockSpec((_B, _D), lambda i: (i, 0)),
            pl.BlockSpec((_B,), lambda i: (i,)),
            pl.BlockSpec((1, 1), lambda i: (0, 0), memory_space=pltpu.SMEM),
        ],
        out_shape=[
            jax.ShapeDtypeStruct((n, _D), jnp.float32),
            jax.ShapeDtypeStruct((n,), jnp.int32),
            jax.ShapeDtypeStruct((1, 1), jnp.float32),
        ],
    )(x, embedding)
    return q, idx, loss


def _sc_gather(embedding, idx):
    idx3 = idx.reshape(_NW, _CPW, _CHUNK)
    mesh = plsc.VectorSubcoreMesh(core_axis_name="c", subcore_axis_name="s")

    @functools.partial(
        pl.kernel,
        mesh=mesh,
        out_type=jax.ShapeDtypeStruct((_NW, _CPW, _CHUNK, _D), jnp.float32),
        scratch_types=[
            pltpu.VMEM((_CPW, _CHUNK), jnp.int32),
            pltpu.VMEM((8, _CHUNK, _D), jnp.float32),
            pltpu.SemaphoreType.DMA,
        ],
    )
    def gather_k(e_hbm, idx_hbm, out_hbm, idx_v, rows_v, sem):
        wid = lax.axis_index("c") * _NS + lax.axis_index("s")
        pltpu.sync_copy(idx_hbm.at[wid], idx_v)
        for half in range(2):
            handles = []
            for b in range(8):
                j = half * 8 + b
                handles.append(
                    pltpu.async_copy(e_hbm.at[idx_v.at[j]], rows_v.at[b], sem)
                )
            for h in handles:
                h.wait()
            for b in range(8):
                j = half * 8 + b
                pltpu.sync_copy(rows_v.at[b], out_hbm.at[wid, j])

    out = gather_k(embedding, idx3)
    return out.reshape(_NW * _CPW * _CHUNK, _D)


def kernel(inputs, embedding):
    x = inputs.reshape(-1, _D)
    q, idx, loss = _tc_part(x, embedding)
    return q, _COMMITMENT_COST * loss[0, 0], idx


# speedup vs baseline: 1.0514x; 1.0514x over previous
import functools

import jax
import jax.numpy as jnp
from jax import lax
from jax.experimental import pallas as pl
from jax.experimental.pallas import tpu as pltpu
from jax.experimental.pallas import tpu_sc as plsc

_K = 1024
_D = 64
_B = 2048
_COMMITMENT_COST = 1.0

_NC = 2
_NS = 16
_NW = _NC * _NS
_CHUNK = 128
_CPW = 16


def _vq_tc(x_ref, e_ref, q_ref, idx_ref, loss_ref):
    i = pl.program_id(0)
    x = x_ref[:]
    e = e_ref[:]
    xn = jnp.sum(x * x, axis=1, keepdims=True)
    en = jnp.sum(e * e, axis=1)
    prod = jax.lax.dot_general(
        x, e, (((1,), (1,)), ((), ())), preferred_element_type=jnp.float32
    )
    dist = xn + en[None, :] - 2.0 * prod
    idx = jnp.argmin(dist, axis=1).astype(jnp.int32)
    idx_ref[:] = idx
    iota = jax.lax.broadcasted_iota(jnp.int32, (_B, _K), 1)
    oh = (idx[:, None] == iota).astype(jnp.float32)
    q = jax.lax.dot_general(
        oh, e, (((1,), (0,)), ((), ())), preferred_element_type=jnp.float32
    )
    q_ref[:] = q

    part = jnp.sum((x - q) ** 2)

    @pl.when(i == 0)
    def _init():
        loss_ref[0, 0] = 0.0

    loss_ref[0, 0] += part

    @pl.when(i == pl.num_programs(0) - 1)
    def _fini():
        loss_ref[0, 0] = loss_ref[0, 0] / (pl.num_programs(0) * _B * _D)


def _tc_part(x, embedding):
    n = x.shape[0]
    q, idx, loss = pl.pallas_call(
        _vq_tc,
        grid=(n // _B,),
        in_specs=[
            pl.BlockSpec((_B, _D), lambda i: (i, 0)),
            pl.BlockSpec((_K, _D), lambda i: (0, 0)),
        ],
        out_specs=[
            pl.BlockSpec((_B, _D), lambda i: (i, 0)),
            pl.BlockSpec((_B,), lambda i: (i,)),
            pl.BlockSpec((1, 1), lambda i: (0, 0), memory_space=pltpu.SMEM),
        ],
        out_shape=[
            jax.ShapeDtypeStruct((n, _D), jnp.float32),
            jax.ShapeDtypeStruct((n,), jnp.int32),
            jax.ShapeDtypeStruct((1, 1), jnp.float32),
        ],
    )(x, embedding)
    return q, idx, loss


def _sc_gather(embedding, idx):
    idx3 = idx.reshape(_NW, _CPW, _CHUNK)
    mesh = plsc.VectorSubcoreMesh(core_axis_name="c", subcore_axis_name="s")

    @functools.partial(
        pl.kernel,
        mesh=mesh,
        out_type=jax.ShapeDtypeStruct((_NW, _CPW, _CHUNK, _D), jnp.float32),
        scratch_types=[
            pltpu.VMEM((_CPW, _CHUNK), jnp.int32),
            pltpu.VMEM((8, _CHUNK, _D), jnp.float32),
            pltpu.SemaphoreType.DMA,
        ],
    )
    def gather_k(e_hbm, idx_hbm, out_hbm, idx_v, rows_v, sem):
        wid = lax.axis_index("c") * _NS + lax.axis_index("s")
        pltpu.sync_copy(idx_hbm.at[wid], idx_v)
        for half in range(2):
            handles = []
            for b in range(8):
                j = half * 8 + b
                handles.append(
                    pltpu.async_copy(e_hbm.at[idx_v.at[j]], rows_v.at[b], sem)
                )
            for h in handles:
                h.wait()
            for b in range(8):
                j = half * 8 + b
                pltpu.sync_copy(rows_v.at[b], out_hbm.at[wid, j])

    out = gather_k(embedding, idx3)
    return out.reshape(_NW * _CPW * _CHUNK, _D)


def kernel(inputs, embedding):
    x = inputs.reshape(-1, _D)
    q, idx, loss = _tc_part(x, embedding)
    return q, _COMMITMENT_COST * loss[0, 0], idx


# speedup vs baseline: 1.0751x; 1.0226x over previous
import functools

import jax
import jax.numpy as jnp
from jax import lax
from jax.experimental import pallas as pl
from jax.experimental.pallas import tpu as pltpu
from jax.experimental.pallas import tpu_sc as plsc

_K = 1024
_D = 64
_B = 2048
_COMMITMENT_COST = 1.0

_NC = 2
_NS = 16
_NW = _NC * _NS
_CHUNK = 128
_CPW = 16


def _vq_tc(x_ref, e_ref, q_ref, idx_ref, loss_ref):
    i = pl.program_id(0)
    x = x_ref[:].reshape(_B, _D)
    e = e_ref[:]
    xn = jnp.sum(x * x, axis=1, keepdims=True)
    en = jnp.sum(e * e, axis=1)
    prod = jax.lax.dot_general(
        x, e, (((1,), (1,)), ((), ())), preferred_element_type=jnp.float32
    )
    dist = xn + en[None, :] - 2.0 * prod
    idx = jnp.argmin(dist, axis=1).astype(jnp.int32)
    idx_ref[:] = idx
    iota = jax.lax.broadcasted_iota(jnp.int32, (_B, _K), 1)
    oh = (idx[:, None] == iota).astype(jnp.float32)
    q = jax.lax.dot_general(
        oh, e, (((1,), (0,)), ((), ())), preferred_element_type=jnp.float32
    )
    q_ref[:] = q

    part = jnp.sum((x - q) ** 2)

    @pl.when(i == 0)
    def _init():
        loss_ref[0, 0] = 0.0

    loss_ref[0, 0] += part

    @pl.when(i == pl.num_programs(0) - 1)
    def _fini():
        loss_ref[0, 0] = loss_ref[0, 0] / (pl.num_programs(0) * _B * _D)


def _tc_part(inputs, embedding):
    rows3 = _B // inputs.shape[1]
    n = inputs.shape[0] * inputs.shape[1]
    q, idx, loss = pl.pallas_call(
        _vq_tc,
        grid=(n // _B,),
        in_specs=[
            pl.BlockSpec((rows3, inputs.shape[1], _D), lambda i: (i, 0, 0)),
            pl.BlockSpec((_K, _D), lambda i: (0, 0)),
        ],
        out_specs=[
            pl.BlockSpec((_B, _D), lambda i: (i, 0)),
            pl.BlockSpec((_B,), lambda i: (i,)),
            pl.BlockSpec((1, 1), lambda i: (0, 0), memory_space=pltpu.SMEM),
        ],
        out_shape=[
            jax.ShapeDtypeStruct((n, _D), jnp.float32),
            jax.ShapeDtypeStruct((n,), jnp.int32),
            jax.ShapeDtypeStruct((1, 1), jnp.float32),
        ],
    )(inputs, embedding)
    return q, idx, loss


def _sc_gather(embedding, idx):
    idx3 = idx.reshape(_NW, _CPW, _CHUNK)
    mesh = plsc.VectorSubcoreMesh(core_axis_name="c", subcore_axis_name="s")

    @functools.partial(
        pl.kernel,
        mesh=mesh,
        out_type=jax.ShapeDtypeStruct((_NW, _CPW, _CHUNK, _D), jnp.float32),
        scratch_types=[
            pltpu.VMEM((_CPW, _CHUNK), jnp.int32),
            pltpu.VMEM((8, _CHUNK, _D), jnp.float32),
            pltpu.SemaphoreType.DMA,
        ],
    )
    def gather_k(e_hbm, idx_hbm, out_hbm, idx_v, rows_v, sem):
        wid = lax.axis_index("c") * _NS + lax.axis_index("s")
        pltpu.sync_copy(idx_hbm.at[wid], idx_v)
        for half in range(2):
            handles = []
            for b in range(8):
                j = half * 8 + b
                handles.append(
                    pltpu.async_copy(e_hbm.at[idx_v.at[j]], rows_v.at[b], sem)
                )
            for h in handles:
                h.wait()
            for b in range(8):
                j = half * 8 + b
                pltpu.sync_copy(rows_v.at[b], out_hbm.at[wid, j])

    out = gather_k(embedding, idx3)
    return out.reshape(_NW * _CPW * _CHUNK, _D)


def kernel(inputs, embedding):
    q, idx, loss = _tc_part(inputs, embedding)
    return q, _COMMITMENT_COST * loss[0, 0], idx
